# Initial kernel scaffold; baseline (speedup 1.0000x reference)
import jax
import jax.numpy as jnp
from jax.experimental import pallas as pl


def kernel(query_xyz, support_xyz, features):
    raise NotImplementedError("write your pallas kernel here")


# speedup vs baseline: 36.8045x; 36.8045x over previous
import dataclasses
import functools

import jax
import jax.numpy as jnp
import numpy as np
from jax import lax
from jax.experimental import pallas as pl
from jax.experimental.pallas import tpu as pltpu
from jax.experimental.pallas import tpu_sc as plsc

RADIUS2 = 0.25 * 0.25
NSAMPLE = 32

NC = 2
NS = 16
NW = NC * NS
L = 16


def _make_pack_mats(n):
  nn = np.arange(n)
  k = np.arange(n // 32)
  blk = (nn[:, None] // 32) == k[None, :]
  w = 2.0 ** (nn % 16)
  slo = (blk * np.where(nn % 32 < 16, w, 0.0)[:, None]).astype(np.float32)
  shi = (blk * np.where(nn % 32 >= 16, w, 0.0)[:, None]).astype(np.float32)
  return jnp.asarray(slo, jnp.bfloat16), jnp.asarray(shi, jnp.bfloat16)


def _maskpack_body(q_ref, st_ref, slo_ref, shi_ref, l1_ref, *, n, n_chunk):
  q = q_ref[0]
  qx, qy, qz = q[:, 0:1], q[:, 1:2], q[:, 2:3]
  q2 = (qx * qx + qy * qy) + qz * qz
  qb = q.astype(jnp.bfloat16)
  lo = None
  hi = None
  for c in range(n // n_chunk):
    sl = pl.ds(c * n_chunk, n_chunk)
    sx = st_ref[0, 0:1, sl]
    sy = st_ref[0, 1:2, sl]
    sz = st_ref[0, 2:3, sl]
    x2 = (sx * sx + sy * sy) + sz * sz
    sb = st_ref[0, :, sl].astype(jnp.bfloat16)
    dot = lax.dot_general(qb, sb, (((1,), (0,)), ((), ())),
                          preferred_element_type=jnp.float32)
    d2 = (q2 + x2) - 2.0 * dot
    m = (d2 <= RADIUS2).astype(jnp.bfloat16)
    dn = (((1,), (0,)), ((), ()))
    plo = lax.dot_general(m, slo_ref[sl, :], dn,
                          preferred_element_type=jnp.float32)
    phi = lax.dot_general(m, shi_ref[sl, :], dn,
                          preferred_element_type=jnp.float32)
    lo = plo if lo is None else lo + plo
    hi = phi if hi is None else hi + phi
  l1_ref[0] = lo.astype(jnp.int32) | (hi.astype(jnp.int32) << 16)


def _maskpack(query_xyz, support_t):
  b, q, _ = query_xyz.shape
  n = support_t.shape[2]
  qt = 256
  nw = n // 32
  slo, shi = _make_pack_mats(n)
  body = functools.partial(_maskpack_body, n=n, n_chunk=2048)
  return pl.pallas_call(
      body,
      grid=(b, q // qt),
      in_specs=[
          pl.BlockSpec((1, qt, 3), lambda i, j: (i, j, 0)),
          pl.BlockSpec((1, 3, n), lambda i, j: (i, 0, 0)),
          pl.BlockSpec((n, nw), lambda i, j: (0, 0)),
          pl.BlockSpec((n, nw), lambda i, j: (0, 0)),
      ],
      out_specs=pl.BlockSpec((1, qt, nw), lambda i, j: (i, j, 0)),
      out_shape=jax.ShapeDtypeStruct((b, q, nw), jnp.int32),
  )(query_xyz, support_t, slo, shi)


def _sc_params():
  cp = pltpu.CompilerParams()
  if "needs_layout_passes" in pltpu.CompilerParams.__dataclass_fields__:
    cp = dataclasses.replace(cp, needs_layout_passes=False)
  return cp


def _iota16():
  return lax.broadcasted_iota(jnp.int32, (L,), 0)


def _splat(x):
  return jnp.broadcast_to(x, (L,))


def _ballfinish(l1):
  bq, nwords = l1.shape
  per_w = bq // NW
  grp = 64
  mesh = plsc.VectorSubcoreMesh(core_axis_name="c", subcore_axis_name="s")

  @functools.partial(
      pl.kernel,
      out_type=jax.ShapeDtypeStruct((bq * NSAMPLE,), jnp.int32),
      mesh=mesh,
      scratch_types=[
          pltpu.VMEM((grp, nwords), jnp.int32),
          pltpu.VMEM((nwords + L,), jnp.int32),
          pltpu.VMEM((64,), jnp.int32),
          pltpu.VMEM((grp * NSAMPLE,), jnp.int32),
      ],
      compiler_params=_sc_params(),
  )
  def k(l1_hbm, idx_hbm, l1_vm, nz_vm, hits_vm, out_vm):
    wid = lax.axis_index("s") * NC + lax.axis_index("c")
    qbase = wid * per_w
    iota = _iota16()

    @pl.loop(0, per_w // grp)
    def _group(g):
      qstart = qbase + g * grp
      pltpu.sync_copy(l1_hbm.at[pl.ds(qstart, grp)], l1_vm)

      @pl.loop(0, grp)
      def _query(qi):
        hits_vm[pl.ds(0, L)] = jnp.zeros((L,), jnp.int32)
        nnz_v = jnp.zeros((L,), jnp.int32)
        for t in range(nwords // L):
          v = l1_vm[qi, pl.ds(t * L, L)]
          m = v != 0
          cnt = m.astype(jnp.int32)
          pfx = plsc.cumsum(cnt)
          slots = jnp.where(m, nnz_v + pfx - 1, 0)
          plsc.store_scatter(nz_vm, [slots], t * L + iota, mask=m)
          nnz_v = nnz_v + plsc.all_reduce_population_count(m)
        nnz = jnp.max(nnz_v)

        def word_body(i, hcnt_v):
          kword = plsc.load_gather(nz_vm, [_splat(i)])
          wv = plsc.load_gather(l1_vm, [_splat(qi), kword])
          out = hcnt_v
          for half in range(2):
            bits = lax.shift_right_logical(wv, iota + half * L) & 1
            bm = bits == 1
            pfx = plsc.cumsum(bits)
            slots = out + pfx - 1
            wm = bm & (slots < 48)
            slots = jnp.where(wm, slots, 0)
            ids = kword * 32 + half * L + iota
            plsc.store_scatter(hits_vm, [slots], ids, mask=wm)
            out = out + plsc.all_reduce_population_count(bm)
          return out

        hcnt_v = lax.fori_loop(0, nnz, word_body, jnp.zeros((L,), jnp.int32))
        m_tot = jnp.minimum(jnp.max(hcnt_v), NSAMPLE)
        sel0 = jnp.where(iota < m_tot, iota, 0)
        sel1 = jnp.where(iota + L < m_tot, iota + L, 0)
        out_vm[pl.ds(qi * NSAMPLE, L)] = plsc.load_gather(hits_vm, [sel0])
        out_vm[pl.ds(qi * NSAMPLE + L, L)] = plsc.load_gather(hits_vm, [sel1])

      pltpu.sync_copy(out_vm, idx_hbm.at[pl.ds(qstart * NSAMPLE,
                                               grp * NSAMPLE)])

  return k(l1)


def _gather(features, support_t, query_t, idx):
  b, c, n = features.shape
  q = query_t.shape[2]
  qs = q * NSAMPLE
  cg = 8
  ch = 1024
  n_ftask = b * (c // cg)
  mesh = plsc.VectorSubcoreMesh(core_axis_name="c", subcore_axis_name="s")

  @functools.partial(
      pl.kernel,
      out_type=(jax.ShapeDtypeStruct((b, c, qs), jnp.float32),
                jax.ShapeDtypeStruct((b, 3, qs), jnp.float32)),
      mesh=mesh,
      scratch_types=[
          pltpu.VMEM((cg, n), jnp.float32),
          pltpu.VMEM((ch,), jnp.int32),
          pltpu.VMEM((cg, ch), jnp.float32),
          pltpu.VMEM((4, q), jnp.float32),
      ],
      compiler_params=_sc_params(),
  )
  def k(f_hbm, st_hbm, qt_hbm, idx_hbm, of_hbm, ox_hbm,
        tab_vm, idx_vm, out_vm, q_vm):
    wid = lax.axis_index("s") * NC + lax.axis_index("c")
    iota = _iota16()

    def run_chunks(body):
      @pl.loop(0, qs // ch)
      def _chunk(s):
        body(s)

    def feature_task(tid):
      tb = tid // (c // cg)
      tc = (tid % (c // cg)) * cg
      pltpu.sync_copy(f_hbm.at[tb, pl.ds(tc, cg)], tab_vm)

      def body(s):
        pltpu.sync_copy(idx_hbm.at[pl.ds(tb * qs + s * ch, ch)], idx_vm)

        @pl.loop(0, ch // L)
        def _vec(v):
          ivec = idx_vm[pl.ds(v * L, L)]
          for cc in range(cg):
            g = plsc.load_gather(tab_vm, [_splat(cc), ivec])
            out_vm[cc, pl.ds(v * L, L)] = g

        pltpu.sync_copy(out_vm,
                        of_hbm.at[tb, pl.ds(tc, cg), pl.ds(s * ch, ch)])

      run_chunks(body)

    def xyz_task(tb):
      pltpu.sync_copy(st_hbm.at[tb], tab_vm.at[pl.ds(0, 3)])
      pltpu.sync_copy(qt_hbm.at[tb], q_vm.at[pl.ds(0, 3)])

      def body(s):
        pltpu.sync_copy(idx_hbm.at[pl.ds(tb * qs + s * ch, ch)], idx_vm)

        @pl.loop(0, ch // L)
        def _vec(v):
          ivec = idx_vm[pl.ds(v * L, L)]
          qpos = lax.shift_right_logical(_splat(s * ch + v * L) + iota, 5)
          for d in range(3):
            g = plsc.load_gather(tab_vm, [_splat(d), ivec])
            qv = plsc.load_gather(q_vm, [_splat(d), qpos])
            out_vm[d, pl.ds(v * L, L)] = g - qv

        pltpu.sync_copy(out_vm.at[pl.ds(0, 3)],
                        ox_hbm.at[tb, :, pl.ds(s * ch, ch)])

      run_chunks(body)

    for r in range(2):
      feature_task(r * NW + wid)

    @pl.when(wid < b)
    def _():
      xyz_task(wid)

  return k(features, support_t, query_t, idx)


def kernel(query_xyz, support_xyz, features):
  b, q, _ = query_xyz.shape
  n = support_xyz.shape[1]
  c = features.shape[1]
  support_t = jnp.transpose(support_xyz, (0, 2, 1))
  query_t = jnp.transpose(query_xyz, (0, 2, 1))
  l1 = _maskpack(query_xyz, support_t)
  idx = _ballfinish(l1.reshape(b * q, n // 32))
  feat_g, xyz_g = _gather(features, support_t, query_t, idx)
  grouped_xyz = xyz_g.reshape(b, 3, q, NSAMPLE)
  grouped_features = feat_g.reshape(b, c, q, NSAMPLE)
  return (grouped_xyz, grouped_features)


# speedup vs baseline: 41.2690x; 1.1213x over previous
import dataclasses
import functools

import jax
import jax.numpy as jnp
import numpy as np
from jax import lax
from jax.experimental import pallas as pl
from jax.experimental.pallas import tpu as pltpu
from jax.experimental.pallas import tpu_sc as plsc

RADIUS2 = 0.25 * 0.25
NSAMPLE = 32

NC = 2
NS = 16
NW = NC * NS
L = 16

NCHUNK = 2048
WCHUNK = NCHUNK // 32


def _make_pack_mats():
  nn = np.arange(NCHUNK)
  k = np.arange(WCHUNK)
  blk = (nn[:, None] // 32) == k[None, :]
  w = 2.0 ** (nn % 16)
  slo = (blk * np.where(nn % 32 < 16, w, 0.0)[:, None]).astype(np.float32)
  shi = (blk * np.where(nn % 32 >= 16, w, 0.0)[:, None]).astype(np.float32)
  grp = ((np.arange(WCHUNK)[:, None] // 16) ==
         np.arange(4)[None, :]).astype(np.float32)
  return (jnp.asarray(slo, jnp.bfloat16), jnp.asarray(shi, jnp.bfloat16),
          jnp.asarray(grp, jnp.bfloat16))


def _maskpack_body(q_ref, st_ref, slo_ref, shi_ref, grp_ref, l1_ref, l2_ref,
                   *, n):
  q = q_ref[0]
  qx, qy, qz = q[:, 0:1], q[:, 1:2], q[:, 2:3]
  q2 = (qx * qx + qy * qy) + qz * qz
  qb = q.astype(jnp.bfloat16)
  dn = (((1,), (0,)), ((), ()))
  l2acc = None
  w4 = (jnp.int32(1) << lax.broadcasted_iota(jnp.int32, (1, 4), 1)
        ).astype(jnp.float32)
  for c in range(n // NCHUNK):
    sl = pl.ds(c * NCHUNK, NCHUNK)
    sx = st_ref[0, 0:1, sl]
    sy = st_ref[0, 1:2, sl]
    sz = st_ref[0, 2:3, sl]
    x2 = (sx * sx + sy * sy) + sz * sz
    sb = st_ref[0, :, sl].astype(jnp.bfloat16)
    dot = lax.dot_general(qb, sb, dn, preferred_element_type=jnp.float32)
    d2 = (q2 + x2) - 2.0 * dot
    m = (d2 <= RADIUS2).astype(jnp.bfloat16)
    plo = lax.dot_general(m, slo_ref[...], dn,
                          preferred_element_type=jnp.float32)
    phi = lax.dot_general(m, shi_ref[...], dn,
                          preferred_element_type=jnp.float32)
    comb = plo.astype(jnp.int32) | (phi.astype(jnp.int32) << 16)
    l1_ref[:, pl.ds(c * WCHUNK, WCHUNK)] = comb
    nzw = (comb != 0).astype(jnp.bfloat16)
    gcnt = lax.dot_general(nzw, grp_ref[...], dn,
                           preferred_element_type=jnp.float32)
    gbit = jnp.where(gcnt > 0, w4, 0.0) * (2.0 ** (4 * c))
    contrib = jnp.sum(gbit, axis=1, keepdims=True)
    l2acc = contrib if l2acc is None else l2acc + contrib
  l2_ref[...] = l2acc.astype(jnp.int32)


def _maskpack(query_xyz, support_t):
  b, q, _ = query_xyz.shape
  n = support_t.shape[2]
  qt = 256
  nw = n // 32
  slo, shi, grp = _make_pack_mats()
  body = functools.partial(_maskpack_body, n=n)
  nrow = q // qt
  return pl.pallas_call(
      body,
      grid=(b, nrow),
      in_specs=[
          pl.BlockSpec((1, qt, 3), lambda i, j: (i, j, 0)),
          pl.BlockSpec((1, 3, n), lambda i, j: (i, 0, 0)),
          pl.BlockSpec((NCHUNK, WCHUNK), lambda i, j: (0, 0)),
          pl.BlockSpec((NCHUNK, WCHUNK), lambda i, j: (0, 0)),
          pl.BlockSpec((WCHUNK, 4), lambda i, j: (0, 0)),
      ],
      out_specs=[
          pl.BlockSpec((qt, nw), lambda i, j: (i * nrow + j, 0)),
          pl.BlockSpec((qt, 1), lambda i, j: (i * nrow + j, 0)),
      ],
      out_shape=[
          jax.ShapeDtypeStruct((b * q, nw), jnp.int32),
          jax.ShapeDtypeStruct((b * q, 1), jnp.int32),
      ],
  )(query_xyz, support_t, slo, shi, grp)


def _sc_params():
  cp = pltpu.CompilerParams()
  if "needs_layout_passes" in pltpu.CompilerParams.__dataclass_fields__:
    cp = dataclasses.replace(cp, needs_layout_passes=False)
  return cp


def _iota16():
  return lax.broadcasted_iota(jnp.int32, (L,), 0)


def _splat(x):
  return jnp.broadcast_to(x, (L,))


def _ballfinish(l1, l2):
  bq, nwords = l1.shape
  per_w = bq // NW
  grp = 64
  mesh = plsc.VectorSubcoreMesh(core_axis_name="c", subcore_axis_name="s")

  @functools.partial(
      pl.kernel,
      out_type=jax.ShapeDtypeStruct((bq * NSAMPLE,), jnp.int32),
      mesh=mesh,
      scratch_types=[
          pltpu.VMEM((grp, nwords), jnp.int32),
          pltpu.VMEM((grp, 1), jnp.int32),
          pltpu.VMEM((L,), jnp.int32),
          pltpu.VMEM((nwords + L,), jnp.int32),
          pltpu.VMEM((64,), jnp.int32),
          pltpu.VMEM((grp * NSAMPLE,), jnp.int32),
      ],
      compiler_params=_sc_params(),
  )
  def k(l1_hbm, l2_hbm, idx_hbm, l1_vm, l2_vm, nzg_vm, nz_vm, hits_vm,
        out_vm):
    wid = lax.axis_index("s") * NC + lax.axis_index("c")
    qbase = wid * per_w
    iota = _iota16()
    zero16 = jnp.zeros((L,), jnp.int32)

    @pl.loop(0, per_w // grp)
    def _group(g):
      qstart = qbase + g * grp
      pltpu.sync_copy(l1_hbm.at[pl.ds(qstart, grp)], l1_vm)
      pltpu.sync_copy(l2_hbm.at[pl.ds(qstart, grp)], l2_vm)

      @pl.loop(0, grp)
      def _query(qi):
        hits_vm[pl.ds(0, L)] = zero16
        l2v = plsc.load_gather(l2_vm, [_splat(qi), zero16])
        gb = (lax.shift_right_logical(l2v, iota) & 1) == 1
        gpfx = plsc.cumsum(gb.astype(jnp.int32))
        gslots = jnp.where(gb, gpfx - 1, 0)
        plsc.store_scatter(nzg_vm, [gslots], iota, mask=gb)
        ngrp = jnp.max(plsc.all_reduce_population_count(gb))

        def grp_body(i, nnz_v):
          gid = plsc.load_gather(nzg_vm, [_splat(i)])
          wvec = plsc.load_gather(l1_vm, [_splat(qi), gid * 16 + iota])
          m = wvec != 0
          pfx = plsc.cumsum(m.astype(jnp.int32))
          slots = jnp.where(m, nnz_v + pfx - 1, 0)
          plsc.store_scatter(nz_vm, [slots], gid * 16 + iota, mask=m)
          return nnz_v + plsc.all_reduce_population_count(m)

        nnz = jnp.max(lax.fori_loop(0, ngrp, grp_body, zero16))

        def word_body(i, hcnt_v):
          kword = plsc.load_gather(nz_vm, [_splat(i)])
          wv = plsc.load_gather(l1_vm, [_splat(qi), kword])
          out = hcnt_v
          for half in range(2):
            bits = lax.shift_right_logical(wv, iota + half * L) & 1
            bm = bits == 1
            pfx = plsc.cumsum(bits)
            slots = out + pfx - 1
            wm = bm & (slots < 48)
            slots = jnp.where(wm, slots, 0)
            ids = kword * 32 + half * L + iota
            plsc.store_scatter(hits_vm, [slots], ids, mask=wm)
            out = out + plsc.all_reduce_population_count(bm)
          return out

        hcnt_v = lax.fori_loop(0, nnz, word_body, zero16)
        m_tot = jnp.minimum(jnp.max(hcnt_v), NSAMPLE)
        sel0 = jnp.where(iota < m_tot, iota, 0)
        sel1 = jnp.where(iota + L < m_tot, iota + L, 0)
        out_vm[pl.ds(qi * NSAMPLE, L)] = plsc.load_gather(hits_vm, [sel0])
        out_vm[pl.ds(qi * NSAMPLE + L, L)] = plsc.load_gather(hits_vm, [sel1])

      pltpu.sync_copy(out_vm, idx_hbm.at[pl.ds(qstart * NSAMPLE,
                                               grp * NSAMPLE)])

  return k(l1, l2)


def _gather(features, support_t, query_t, idx):
  b, c, n = features.shape
  q = query_t.shape[2]
  qs = q * NSAMPLE
  cg = 8
  ch = 2048
  nch = qs // ch
  mesh = plsc.VectorSubcoreMesh(core_axis_name="c", subcore_axis_name="s")

  @functools.partial(
      pl.kernel,
      out_type=(jax.ShapeDtypeStruct((b, c, qs), jnp.float32),
                jax.ShapeDtypeStruct((b, 3, qs), jnp.float32)),
      mesh=mesh,
      scratch_types=[
          pltpu.VMEM((cg, n), jnp.float32),
          pltpu.VMEM((2, ch), jnp.int32),
          pltpu.VMEM((2, cg, ch), jnp.float32),
          pltpu.VMEM((4, q // 2), jnp.float32),
          pltpu.SemaphoreType.DMA,
          pltpu.SemaphoreType.DMA,
          pltpu.SemaphoreType.DMA,
          pltpu.SemaphoreType.DMA,
      ],
      compiler_params=_sc_params(),
  )
  def k(f_hbm, st_hbm, qt_hbm, idx_hbm, of_hbm, ox_hbm,
        tab_vm, idx_vm, out_vm, q_vm, si0, si1, so0, so1):
    wid = lax.axis_index("s") * NC + lax.axis_index("c")
    iota = _iota16()
    si = (si0, si1)
    so = (so0, so1)

    def idx_cp(tb, s, bi):
      return pltpu.make_async_copy(
          idx_hbm.at[pl.ds(tb * qs + s * ch, ch)], idx_vm.at[bi], si[bi])

    def ring(tb, s_lo, s_hi, compute, out_cp):
      idx_cp(tb, s_lo, 0).start()
      idx_cp(tb, s_lo + 1, 1).start()

      @pl.loop(0, (s_hi - s_lo) // 2)
      def _pair(g):
        for bi in range(2):
          s = s_lo + g * 2 + bi
          idx_cp(tb, s, bi).wait()

          @pl.when(g > 0)
          def _():
            out_cp(tb, s - 2, bi).wait()

          compute(s, bi)
          out_cp(tb, s, bi).start()

          @pl.when(s + 2 < s_hi)
          def _():
            idx_cp(tb, s + 2, bi).start()

      for bi in range(2):
        out_cp(tb, s_hi - 2 + bi, bi).wait()

    def feature_task(tid):
      tb = tid // (c // cg)
      tc = (tid % (c // cg)) * cg
      pltpu.sync_copy(f_hbm.at[tb, pl.ds(tc, cg)], tab_vm)

      def compute(s, bi):
        @pl.loop(0, ch // L)
        def _vec(v):
          ivec = idx_vm[bi, pl.ds(v * L, L)]
          for cc in range(cg):
            g = plsc.load_gather(tab_vm, [_splat(cc), ivec])
            out_vm[bi, cc, pl.ds(v * L, L)] = g

      def out_cp(tb_, s, bi):
        return pltpu.make_async_copy(
            out_vm.at[bi],
            of_hbm.at[tb_, pl.ds(tc, cg), pl.ds(s * ch, ch)], so[bi])

      ring(tb, 0, nch, compute, out_cp)

    def xyz_task(xt):
      tb = xt // 2
      h = xt % 2
      pltpu.sync_copy(st_hbm.at[tb], tab_vm.at[pl.ds(0, 3)])
      pltpu.sync_copy(qt_hbm.at[tb, :, pl.ds(h * (q // 2), q // 2)],
                      q_vm.at[pl.ds(0, 3)])
      qoff = h * (q // 2)

      def compute(s, bi):
        @pl.loop(0, ch // L)
        def _vec(v):
          ivec = idx_vm[bi, pl.ds(v * L, L)]
          qpos = lax.shift_right_logical(_splat(s * ch + v * L) + iota,
                                         5) - qoff
          for d in range(3):
            g = plsc.load_gather(tab_vm, [_splat(d), ivec])
            qv = plsc.load_gather(q_vm, [_splat(d), qpos])
            out_vm[bi, d, pl.ds(v * L, L)] = g - qv

      def out_cp(tb_, s, bi):
        return pltpu.make_async_copy(
            out_vm.at[bi, pl.ds(0, 3)],
            ox_hbm.at[tb_, :, pl.ds(s * ch, ch)], so[bi])

      ring(tb, h * (nch // 2), (h + 1) * (nch // 2), compute, out_cp)

    for r in range(2):
      feature_task(r * NW + wid)

    @pl.when(wid >= NW - 2 * b)
    def _():
      xyz_task(wid - (NW - 2 * b))

  return k(features, support_t, query_t, idx)


def kernel(query_xyz, support_xyz, features):
  b, q, _ = query_xyz.shape
  n = support_xyz.shape[1]
  c = features.shape[1]
  support_t = jnp.transpose(support_xyz, (0, 2, 1))
  query_t = jnp.transpose(query_xyz, (0, 2, 1))
  l1, l2 = _maskpack(query_xyz, support_t)
  idx = _ballfinish(l1, l2)
  feat_g, xyz_g = _gather(features, support_t, query_t, idx)
  grouped_xyz = xyz_g.reshape(b, 3, q, NSAMPLE)
  grouped_features = feat_g.reshape(b, c, q, NSAMPLE)
  return (grouped_xyz, grouped_features)


# speedup vs baseline: 41.3216x; 1.0013x over previous
import dataclasses
import functools

import jax
import jax.numpy as jnp
import numpy as np
from jax import lax
from jax.experimental import pallas as pl
from jax.experimental.pallas import tpu as pltpu
from jax.experimental.pallas import tpu_sc as plsc

RADIUS2 = 0.25 * 0.25
NSAMPLE = 32

NC = 2
NS = 16
NW = NC * NS
L = 16

NCHUNK = 2048
WCHUNK = NCHUNK // 32


def _make_pack_mats():
  nn = np.arange(NCHUNK)
  k = np.arange(WCHUNK)
  blk = (nn[:, None] // 32) == k[None, :]
  w = 2.0 ** (nn % 16)
  slo = (blk * np.where(nn % 32 < 16, w, 0.0)[:, None]).astype(np.float32)
  shi = (blk * np.where(nn % 32 >= 16, w, 0.0)[:, None]).astype(np.float32)
  grp = ((np.arange(WCHUNK)[:, None] // 16) ==
         np.arange(4)[None, :]).astype(np.float32)
  return (jnp.asarray(slo, jnp.bfloat16), jnp.asarray(shi, jnp.bfloat16),
          jnp.asarray(grp, jnp.bfloat16))


def _maskpack_body(q_ref, st_ref, slo_ref, shi_ref, grp_ref, l1_ref, l2_ref,
                   *, n):
  q = q_ref[0]
  qx, qy, qz = q[:, 0:1], q[:, 1:2], q[:, 2:3]
  q2 = (qx * qx + qy * qy) + qz * qz
  qb = q.astype(jnp.bfloat16)
  dn = (((1,), (0,)), ((), ()))
  l2acc = None
  w4 = (jnp.int32(1) << lax.broadcasted_iota(jnp.int32, (1, 4), 1)
        ).astype(jnp.float32)
  for c in range(n // NCHUNK):
    sl = pl.ds(c * NCHUNK, NCHUNK)
    sx = st_ref[0, 0:1, sl]
    sy = st_ref[0, 1:2, sl]
    sz = st_ref[0, 2:3, sl]
    x2 = (sx * sx + sy * sy) + sz * sz
    sb = st_ref[0, :, sl].astype(jnp.bfloat16)
    dot = lax.dot_general(qb, sb, dn, preferred_element_type=jnp.float32)
    d2 = (q2 + x2) - 2.0 * dot
    m = (d2 <= RADIUS2).astype(jnp.bfloat16)
    plo = lax.dot_general(m, slo_ref[...], dn,
                          preferred_element_type=jnp.float32)
    phi = lax.dot_general(m, shi_ref[...], dn,
                          preferred_element_type=jnp.float32)
    comb = plo.astype(jnp.int32) | (phi.astype(jnp.int32) << 16)
    l1_ref[:, pl.ds(c * WCHUNK, WCHUNK)] = comb
    nzw = (comb != 0).astype(jnp.bfloat16)
    gcnt = lax.dot_general(nzw, grp_ref[...], dn,
                           preferred_element_type=jnp.float32)
    gbit = jnp.where(gcnt > 0, w4, 0.0) * (2.0 ** (4 * c))
    contrib = jnp.sum(gbit, axis=1, keepdims=True)
    l2acc = contrib if l2acc is None else l2acc + contrib
  l2_ref[...] = l2acc.astype(jnp.int32)


def _maskpack(query_xyz, support_t):
  b, q, _ = query_xyz.shape
  n = support_t.shape[2]
  qt = 256
  nw = n // 32
  slo, shi, grp = _make_pack_mats()
  body = functools.partial(_maskpack_body, n=n)
  nrow = q // qt
  return pl.pallas_call(
      body,
      grid=(b, nrow),
      in_specs=[
          pl.BlockSpec((1, qt, 3), lambda i, j: (i, j, 0)),
          pl.BlockSpec((1, 3, n), lambda i, j: (i, 0, 0)),
          pl.BlockSpec((NCHUNK, WCHUNK), lambda i, j: (0, 0)),
          pl.BlockSpec((NCHUNK, WCHUNK), lambda i, j: (0, 0)),
          pl.BlockSpec((WCHUNK, 4), lambda i, j: (0, 0)),
      ],
      out_specs=[
          pl.BlockSpec((qt, nw), lambda i, j: (i * nrow + j, 0)),
          pl.BlockSpec((qt, 1), lambda i, j: (i * nrow + j, 0)),
      ],
      out_shape=[
          jax.ShapeDtypeStruct((b * q, nw), jnp.int32),
          jax.ShapeDtypeStruct((b * q, 1), jnp.int32),
      ],
  )(query_xyz, support_t, slo, shi, grp)


def _sc_params():
  cp = pltpu.CompilerParams()
  if "needs_layout_passes" in pltpu.CompilerParams.__dataclass_fields__:
    cp = dataclasses.replace(cp, needs_layout_passes=False)
  return cp


def _iota16():
  return lax.broadcasted_iota(jnp.int32, (L,), 0)


def _splat(x):
  return jnp.broadcast_to(x, (L,))


def _ballfinish(l1, l2):
  bq, nwords = l1.shape
  per_w = bq // NW
  grp = 64
  mesh = plsc.VectorSubcoreMesh(core_axis_name="c", subcore_axis_name="s")

  @functools.partial(
      pl.kernel,
      out_type=jax.ShapeDtypeStruct((bq * NSAMPLE,), jnp.int32),
      mesh=mesh,
      scratch_types=[
          pltpu.VMEM((grp, nwords), jnp.int32),
          pltpu.VMEM((grp, 1), jnp.int32),
          pltpu.VMEM((L,), jnp.int32),
          pltpu.VMEM((nwords + L,), jnp.int32),
          pltpu.VMEM((64,), jnp.int32),
          pltpu.VMEM((grp * NSAMPLE,), jnp.int32),
      ],
      compiler_params=_sc_params(),
  )
  def k(l1_hbm, l2_hbm, idx_hbm, l1_vm, l2_vm, nzg_vm, nz_vm, hits_vm,
        out_vm):
    wid = lax.axis_index("s") * NC + lax.axis_index("c")
    qbase = wid * per_w
    iota = _iota16()
    zero16 = jnp.zeros((L,), jnp.int32)

    @pl.loop(0, per_w // grp)
    def _group(g):
      qstart = qbase + g * grp
      pltpu.sync_copy(l1_hbm.at[pl.ds(qstart, grp)], l1_vm)
      pltpu.sync_copy(l2_hbm.at[pl.ds(qstart, grp)], l2_vm)

      @pl.loop(0, grp)
      def _query(qi):
        hits_vm[pl.ds(0, L)] = zero16
        l2v = plsc.load_gather(l2_vm, [_splat(qi), zero16])
        gb = (lax.shift_right_logical(l2v, iota) & 1) == 1
        gpfx = plsc.cumsum(gb.astype(jnp.int32))
        gslots = jnp.where(gb, gpfx - 1, 0)
        plsc.store_scatter(nzg_vm, [gslots], iota, mask=gb)
        ngrp = jnp.max(plsc.all_reduce_population_count(gb))

        def grp_body(i, nnz_v):
          gid = plsc.load_gather(nzg_vm, [_splat(i)])
          wvec = plsc.load_gather(l1_vm, [_splat(qi), gid * 16 + iota])
          m = wvec != 0
          pfx = plsc.cumsum(m.astype(jnp.int32))
          slots = jnp.where(m, nnz_v + pfx - 1, 0)
          plsc.store_scatter(nz_vm, [slots], gid * 16 + iota, mask=m)
          return nnz_v + plsc.all_reduce_population_count(m)

        nnz = jnp.max(lax.fori_loop(0, ngrp, grp_body, zero16))

        def word_body(i, hcnt_v):
          kword = plsc.load_gather(nz_vm, [_splat(i)])
          wv = plsc.load_gather(l1_vm, [_splat(qi), kword])
          out = hcnt_v
          for half in range(2):
            bits = lax.shift_right_logical(wv, iota + half * L) & 1
            bm = bits == 1
            pfx = plsc.cumsum(bits)
            slots = out + pfx - 1
            wm = bm & (slots < 48)
            slots = jnp.where(wm, slots, 0)
            ids = kword * 32 + half * L + iota
            plsc.store_scatter(hits_vm, [slots], ids, mask=wm)
            out = out + plsc.all_reduce_population_count(bm)
          return out

        hcnt_v = lax.fori_loop(0, nnz, word_body, zero16)
        m_tot = jnp.minimum(jnp.max(hcnt_v), NSAMPLE)
        sel0 = jnp.where(iota < m_tot, iota, 0)
        sel1 = jnp.where(iota + L < m_tot, iota + L, 0)
        out_vm[pl.ds(qi * NSAMPLE, L)] = plsc.load_gather(hits_vm, [sel0])
        out_vm[pl.ds(qi * NSAMPLE + L, L)] = plsc.load_gather(hits_vm, [sel1])

      pltpu.sync_copy(out_vm, idx_hbm.at[pl.ds(qstart * NSAMPLE,
                                               grp * NSAMPLE)])

  return k(l1, l2)


def _gather(features, support_t, query_t, idx):
  b, c, n = features.shape
  q = query_t.shape[2]
  qs = q * NSAMPLE
  cg = 8
  ch = 2048
  nch = qs // ch
  mesh = plsc.VectorSubcoreMesh(core_axis_name="c", subcore_axis_name="s")

  @functools.partial(
      pl.kernel,
      out_type=(jax.ShapeDtypeStruct((b, c, qs // 128, 128), jnp.float32),
                jax.ShapeDtypeStruct((b, 3, qs // 128, 128), jnp.float32)),
      mesh=mesh,
      scratch_types=[
          pltpu.VMEM((cg, n), jnp.float32),
          pltpu.VMEM((2, ch), jnp.int32),
          pltpu.VMEM((2, cg, ch // 128, 128), jnp.float32),
          pltpu.VMEM((4, q // 2), jnp.float32),
          pltpu.SemaphoreType.DMA,
          pltpu.SemaphoreType.DMA,
          pltpu.SemaphoreType.DMA,
          pltpu.SemaphoreType.DMA,
      ],
      compiler_params=_sc_params(),
  )
  def k(f_hbm, st_hbm, qt_hbm, idx_hbm, of_hbm, ox_hbm,
        tab_vm, idx_vm, out_vm, q_vm, si0, si1, so0, so1):
    wid = lax.axis_index("s") * NC + lax.axis_index("c")
    iota = _iota16()
    si = (si0, si1)
    so = (so0, so1)

    def idx_cp(tb, s, bi):
      return pltpu.make_async_copy(
          idx_hbm.at[pl.ds(tb * qs + s * ch, ch)], idx_vm.at[bi], si[bi])

    def ring(tb, s_lo, s_hi, compute, out_cp):
      idx_cp(tb, s_lo, 0).start()
      idx_cp(tb, s_lo + 1, 1).start()

      @pl.loop(0, (s_hi - s_lo) // 2)
      def _pair(g):
        for bi in range(2):
          s = s_lo + g * 2 + bi
          idx_cp(tb, s, bi).wait()

          @pl.when(g > 0)
          def _():
            out_cp(tb, s - 2, bi).wait()

          compute(s, bi)
          out_cp(tb, s, bi).start()

          @pl.when(s + 2 < s_hi)
          def _():
            idx_cp(tb, s + 2, bi).start()

      for bi in range(2):
        out_cp(tb, s_hi - 2 + bi, bi).wait()

    def feature_task(tid):
      tb = tid // (c // cg)
      tc = (tid % (c // cg)) * cg
      pltpu.sync_copy(f_hbm.at[tb, pl.ds(tc, cg)], tab_vm)

      def compute(s, bi):
        @pl.loop(0, ch // L)
        def _vec(v):
          ivec = idx_vm[bi, pl.ds(v * L, L)]
          for cc in range(cg):
            g = plsc.load_gather(tab_vm, [_splat(cc), ivec])
            out_vm[bi, cc, v // 8, pl.ds((v % 8) * L, L)] = g

      def out_cp(tb_, s, bi):
        rch = ch // 128
        return pltpu.make_async_copy(
            out_vm.at[bi],
            of_hbm.at[tb_, pl.ds(tc, cg), pl.ds(s * rch, rch)], so[bi])

      ring(tb, 0, nch, compute, out_cp)

    def xyz_task(xt):
      tb = xt // 2
      h = xt % 2
      pltpu.sync_copy(st_hbm.at[tb], tab_vm.at[pl.ds(0, 3)])
      pltpu.sync_copy(qt_hbm.at[tb, :, pl.ds(h * (q // 2), q // 2)],
                      q_vm.at[pl.ds(0, 3)])
      qoff = h * (q // 2)

      def compute(s, bi):
        @pl.loop(0, ch // L)
        def _vec(v):
          ivec = idx_vm[bi, pl.ds(v * L, L)]
          qpos = lax.shift_right_logical(_splat(s * ch + v * L) + iota,
                                         5) - qoff
          for d in range(3):
            g = plsc.load_gather(tab_vm, [_splat(d), ivec])
            qv = plsc.load_gather(q_vm, [_splat(d), qpos])
            out_vm[bi, d, v // 8, pl.ds((v % 8) * L, L)] = g - qv

      def out_cp(tb_, s, bi):
        rch = ch // 128
        return pltpu.make_async_copy(
            out_vm.at[bi, pl.ds(0, 3)],
            ox_hbm.at[tb_, :, pl.ds(s * rch, rch)], so[bi])

      ring(tb, h * (nch // 2), (h + 1) * (nch // 2), compute, out_cp)

    for r in range(2):
      feature_task(r * NW + wid)

    @pl.when(wid >= NW - 2 * b)
    def _():
      xyz_task(wid - (NW - 2 * b))

  return k(features, support_t, query_t, idx)


def kernel(query_xyz, support_xyz, features):
  b, q, _ = query_xyz.shape
  n = support_xyz.shape[1]
  c = features.shape[1]
  support_t = jnp.transpose(support_xyz, (0, 2, 1))
  query_t = jnp.transpose(query_xyz, (0, 2, 1))
  l1, l2 = _maskpack(query_xyz, support_t)
  idx = _ballfinish(l1, l2)
  feat_g, xyz_g = _gather(features, support_t, query_t, idx)
  grouped_xyz = xyz_g.reshape(b, 3, q, NSAMPLE)
  grouped_features = feat_g.reshape(b, c, q, NSAMPLE)
  return (grouped_xyz, grouped_features)


# speedup vs baseline: 53.8339x; 1.3028x over previous
import dataclasses
import functools

import jax
import jax.numpy as jnp
import numpy as np
from jax import lax
from jax.experimental import pallas as pl
from jax.experimental.pallas import tpu as pltpu
from jax.experimental.pallas import tpu_sc as plsc

RADIUS2 = 0.25 * 0.25
NSAMPLE = 32

NC = 2
NS = 16
NW = NC * NS
L = 16

NCHUNK = 2048
WCHUNK = NCHUNK // 32


def _make_pack_mats():
  nn = np.arange(NCHUNK)
  k = np.arange(WCHUNK)
  blk = (nn[:, None] // 32) == k[None, :]
  w = 2.0 ** (nn % 16)
  slo = (blk * np.where(nn % 32 < 16, w, 0.0)[:, None]).astype(np.float32)
  shi = (blk * np.where(nn % 32 >= 16, w, 0.0)[:, None]).astype(np.float32)
  grp = ((np.arange(WCHUNK)[:, None] // 16) ==
         np.arange(4)[None, :]).astype(np.float32)
  return (jnp.asarray(slo, jnp.bfloat16), jnp.asarray(shi, jnp.bfloat16),
          jnp.asarray(grp, jnp.bfloat16))


def _maskpack_body(q_ref, st_ref, slo_ref, shi_ref, grp_ref, l1_ref, l2_ref,
                   *, n):
  q = q_ref[0]
  qx, qy, qz = q[:, 0:1], q[:, 1:2], q[:, 2:3]
  q2 = (qx * qx + qy * qy) + qz * qz
  qb = q.astype(jnp.bfloat16)
  dn = (((1,), (0,)), ((), ()))
  l2acc = None
  w4 = (jnp.int32(1) << lax.broadcasted_iota(jnp.int32, (1, 4), 1)
        ).astype(jnp.float32)
  for c in range(n // NCHUNK):
    sl = pl.ds(c * NCHUNK, NCHUNK)
    sx = st_ref[0, 0:1, sl]
    sy = st_ref[0, 1:2, sl]
    sz = st_ref[0, 2:3, sl]
    x2 = (sx * sx + sy * sy) + sz * sz
    sb = st_ref[0, :, sl].astype(jnp.bfloat16)
    dot = lax.dot_general(qb, sb, dn, preferred_element_type=jnp.float32)
    d2 = (q2 + x2) - 2.0 * dot
    m = (d2 <= RADIUS2).astype(jnp.bfloat16)
    plo = lax.dot_general(m, slo_ref[...], dn,
                          preferred_element_type=jnp.float32)
    phi = lax.dot_general(m, shi_ref[...], dn,
                          preferred_element_type=jnp.float32)
    comb = plo.astype(jnp.int32) | (phi.astype(jnp.int32) << 16)
    l1_ref[:, pl.ds(c * WCHUNK, WCHUNK)] = comb
    nzw = (comb != 0).astype(jnp.bfloat16)
    gcnt = lax.dot_general(nzw, grp_ref[...], dn,
                           preferred_element_type=jnp.float32)
    gbit = jnp.where(gcnt > 0, w4, 0.0) * (2.0 ** (4 * c))
    contrib = jnp.sum(gbit, axis=1, keepdims=True)
    l2acc = contrib if l2acc is None else l2acc + contrib
  l2_ref[...] = l2acc.astype(jnp.int32)


def _maskpack(query_xyz, support_t):
  b, q, _ = query_xyz.shape
  n = support_t.shape[2]
  qt = 256
  nw = n // 32
  slo, shi, grp = _make_pack_mats()
  body = functools.partial(_maskpack_body, n=n)
  nrow = q // qt
  return pl.pallas_call(
      body,
      grid=(b, nrow),
      in_specs=[
          pl.BlockSpec((1, qt, 3), lambda i, j: (i, j, 0)),
          pl.BlockSpec((1, 3, n), lambda i, j: (i, 0, 0)),
          pl.BlockSpec((NCHUNK, WCHUNK), lambda i, j: (0, 0)),
          pl.BlockSpec((NCHUNK, WCHUNK), lambda i, j: (0, 0)),
          pl.BlockSpec((WCHUNK, 4), lambda i, j: (0, 0)),
      ],
      out_specs=[
          pl.BlockSpec((qt, nw), lambda i, j: (i * nrow + j, 0)),
          pl.BlockSpec((qt, 1), lambda i, j: (i * nrow + j, 0)),
      ],
      out_shape=[
          jax.ShapeDtypeStruct((b * q, nw), jnp.int32),
          jax.ShapeDtypeStruct((b * q, 1), jnp.int32),
      ],
  )(query_xyz, support_t, slo, shi, grp)


def _sc_params():
  cp = pltpu.CompilerParams()
  if "needs_layout_passes" in pltpu.CompilerParams.__dataclass_fields__:
    cp = dataclasses.replace(cp, needs_layout_passes=False)
  return cp


def _iota16():
  return lax.broadcasted_iota(jnp.int32, (L,), 0)


def _splat(x):
  return jnp.broadcast_to(x, (L,))


IDXW = NSAMPLE + 1


def _ballfinish(l1, l2):
  bq, nwords = l1.shape
  per_w = bq // NW
  grp = 64
  mesh = plsc.VectorSubcoreMesh(core_axis_name="c", subcore_axis_name="s")

  @functools.partial(
      pl.kernel,
      out_type=jax.ShapeDtypeStruct((bq * IDXW,), jnp.int32),
      mesh=mesh,
      scratch_types=[
          pltpu.VMEM((grp, nwords), jnp.int32),
          pltpu.VMEM((grp, 1), jnp.int32),
          pltpu.VMEM((L,), jnp.int32),
          pltpu.VMEM((nwords + L,), jnp.int32),
          pltpu.VMEM((64,), jnp.int32),
          pltpu.VMEM((grp * IDXW,), jnp.int32),
      ],
      compiler_params=_sc_params(),
  )
  def k(l1_hbm, l2_hbm, idx_hbm, l1_vm, l2_vm, nzg_vm, nz_vm, hits_vm,
        out_vm):
    wid = lax.axis_index("s") * NC + lax.axis_index("c")
    qbase = wid * per_w
    iota = _iota16()
    zero16 = jnp.zeros((L,), jnp.int32)

    @pl.loop(0, per_w // grp)
    def _group(g):
      qstart = qbase + g * grp
      pltpu.sync_copy(l1_hbm.at[pl.ds(qstart, grp)], l1_vm)
      pltpu.sync_copy(l2_hbm.at[pl.ds(qstart, grp)], l2_vm)

      @pl.loop(0, grp)
      def _query(qi):
        hits_vm[pl.ds(0, L)] = zero16
        l2v = plsc.load_gather(l2_vm, [_splat(qi), zero16])
        gb = (lax.shift_right_logical(l2v, iota) & 1) == 1
        gpfx = plsc.cumsum(gb.astype(jnp.int32))
        gslots = jnp.where(gb, gpfx - 1, 0)
        plsc.store_scatter(nzg_vm, [gslots], iota, mask=gb)
        ngrp = jnp.max(plsc.all_reduce_population_count(gb))

        def grp_body(i, nnz_v):
          gid = plsc.load_gather(nzg_vm, [_splat(i)])
          wvec = plsc.load_gather(l1_vm, [_splat(qi), gid * 16 + iota])
          m = wvec != 0
          pfx = plsc.cumsum(m.astype(jnp.int32))
          slots = jnp.where(m, nnz_v + pfx - 1, 0)
          plsc.store_scatter(nz_vm, [slots], gid * 16 + iota, mask=m)
          return nnz_v + plsc.all_reduce_population_count(m)

        nnz = jnp.max(lax.fori_loop(0, ngrp, grp_body, zero16))

        def word_body(i, hcnt_v):
          kword = plsc.load_gather(nz_vm, [_splat(i)])
          wv = plsc.load_gather(l1_vm, [_splat(qi), kword])
          out = hcnt_v
          for half in range(2):
            bits = lax.shift_right_logical(wv, iota + half * L) & 1
            bm = bits == 1
            pfx = plsc.cumsum(bits)
            slots = out + pfx - 1
            wm = bm & (slots < 48)
            slots = jnp.where(wm, slots, 0)
            ids = kword * 32 + half * L + iota
            plsc.store_scatter(hits_vm, [slots], ids, mask=wm)
            out = out + plsc.all_reduce_population_count(bm)
          return out

        hcnt_v = lax.fori_loop(0, nnz, word_body, zero16)
        m_tot = jnp.minimum(jnp.max(hcnt_v), NSAMPLE)
        sel0 = jnp.where(iota < m_tot, iota, 0)
        sel1 = jnp.where(iota + L < m_tot, iota + L, 0)
        out_vm[pl.ds(qi * IDXW, L)] = plsc.load_gather(hits_vm, [sel0])
        out_vm[pl.ds(qi * IDXW + L, L)] = plsc.load_gather(hits_vm, [sel1])

      pltpu.sync_copy(out_vm, idx_hbm.at[pl.ds(qstart * IDXW, grp * IDXW)])

  return k(l1, l2)


def _gather(features, support_t, query_t, idx):
  b, c, n = features.shape
  q = query_t.shape[2]
  cg = 8
  qch = 128
  sh = NSAMPLE // 2
  mesh = plsc.VectorSubcoreMesh(core_axis_name="c", subcore_axis_name="s")

  @functools.partial(
      pl.kernel,
      out_type=(jax.ShapeDtypeStruct((b, c, NSAMPLE, q), jnp.float32),
                jax.ShapeDtypeStruct((b, 3, NSAMPLE, q), jnp.float32)),
      mesh=mesh,
      scratch_types=[
          pltpu.VMEM((cg, n), jnp.float32),
          pltpu.VMEM((qch * IDXW,), jnp.int32),
          pltpu.VMEM((qch * IDXW,), jnp.int32),
          pltpu.VMEM((1, cg, sh, qch), jnp.float32),
          pltpu.VMEM((1, cg, sh, qch), jnp.float32),
          pltpu.VMEM((4, q), jnp.float32),
          pltpu.SemaphoreType.DMA,
          pltpu.SemaphoreType.DMA,
          pltpu.SemaphoreType.DMA,
          pltpu.SemaphoreType.DMA,
      ],
      compiler_params=_sc_params(),
  )
  def k(f_hbm, st_hbm, qt_hbm, idx_hbm, of_hbm, ox_hbm,
        tab_vm, idx_a, idx_b, out_a, out_b, q_vm, si0, si1, so0, so1):
    wid = lax.axis_index("s") * NC + lax.axis_index("c")
    iota = _iota16()
    si = (si0, si1)
    so = (so0, so1)
    outs = (out_a, out_b)
    idxs = (idx_a, idx_b)

    def idx_cp(tb, s, bi):
      return pltpu.make_async_copy(
          idx_hbm.at[pl.ds((tb * q + s * qch) * IDXW, qch * IDXW)],
          idxs[bi], si[bi])

    def ring(tb, s_lo, s_hi, compute, out_cp):
      idx_cp(tb, s_lo, 0).start()
      idx_cp(tb, s_lo + 1, 1).start()

      @pl.loop(0, (s_hi - s_lo) // 2)
      def _pair(g):
        for bi in range(2):
          s = s_lo + g * 2 + bi
          idx_cp(tb, s, bi).wait()
          for half in range(2):
            @pl.when(s > s_lo)
            def _():
              out_cp(tb, s - 1, half).wait()

            compute(s, bi, half, outs[half])
            out_cp(tb, s, half).start()

          @pl.when(s + 2 < s_hi)
          def _():
            idx_cp(tb, s + 2, bi).start()

      for half in range(2):
        out_cp(tb, s_hi - 1, half).wait()

    def feature_task(tid):
      tb = tid // (c // cg)
      tc = (tid % (c // cg)) * cg
      pltpu.sync_copy(f_hbm.at[tb, pl.ds(tc, cg)], tab_vm)

      def compute(s, bi, half, ob):
        @pl.loop(0, qch // L)
        def _qb(v):
          qrow = (_splat(v * L) + iota) * IDXW
          for s16 in range(sh):
            ivec = plsc.load_gather(idxs[bi], [qrow + half * sh + s16])
            for cc in range(cg):
              g = plsc.load_gather(tab_vm, [_splat(cc), ivec])
              ob[0, cc, s16, pl.ds(v * L, L)] = g

      def out_cp(tb_, s, half):
        return pltpu.make_async_copy(
            outs[half],
            of_hbm.at[pl.ds(tb_, 1), pl.ds(tc, cg),
                      pl.ds(half * sh, sh), pl.ds(s * qch, qch)],
            so[half])

      ring(tb, 0, q // qch, compute, out_cp)

    def xyz_task(xt):
      tb = xt // 4
      h = xt % 4
      nck = q // qch // 4
      pltpu.sync_copy(st_hbm.at[tb], tab_vm.at[pl.ds(0, 3)])
      pltpu.sync_copy(qt_hbm.at[tb], q_vm.at[pl.ds(0, 3)])

      def compute(s, bi, half, ob):
        @pl.loop(0, qch // L)
        def _qb(v):
          qrow = (_splat(v * L) + iota) * IDXW
          qv = [q_vm[d, pl.ds(s * qch + v * L, L)] for d in range(3)]
          for s16 in range(sh):
            ivec = plsc.load_gather(idxs[bi], [qrow + half * sh + s16])
            for d in range(3):
              g = plsc.load_gather(tab_vm, [_splat(d), ivec])
              ob[0, d, s16, pl.ds(v * L, L)] = g - qv[d]

      def out_cp(tb_, s, half):
        return pltpu.make_async_copy(
            outs[half].at[:, pl.ds(0, 3)],
            ox_hbm.at[pl.ds(tb_, 1), :,
                      pl.ds(half * sh, sh), pl.ds(s * qch, qch)], so[half])

      ring(tb, h * nck, (h + 1) * nck, compute, out_cp)

    for r in range(b * (c // cg) // NW):
      feature_task(r * NW + wid)

    xyz_task(wid)

  return k(features, support_t, query_t, idx)


def kernel(query_xyz, support_xyz, features):
  b, q, _ = query_xyz.shape
  n = support_xyz.shape[1]
  c = features.shape[1]
  support_t = jnp.transpose(support_xyz, (0, 2, 1))
  query_t = jnp.transpose(query_xyz, (0, 2, 1))
  l1, l2 = _maskpack(query_xyz, support_t)
  idx = _ballfinish(l1, l2)
  feat_g, xyz_g = _gather(features, support_t, query_t, idx)
  grouped_xyz = jnp.transpose(xyz_g, (0, 1, 3, 2))
  grouped_features = jnp.transpose(feat_g, (0, 1, 3, 2))
  return (grouped_xyz, grouped_features)


# speedup vs baseline: 61.8606x; 1.1491x over previous
import dataclasses
import functools

import jax
import jax.numpy as jnp
import numpy as np
from jax import lax
from jax.experimental import pallas as pl
from jax.experimental.pallas import tpu as pltpu
from jax.experimental.pallas import tpu_sc as plsc

RADIUS2 = 0.25 * 0.25
NSAMPLE = 32

NC = 2
NS = 16
NW = NC * NS
L = 16

NCHUNK = 2048
WCHUNK = NCHUNK // 32


def _make_pack_mats():
  nn = np.arange(NCHUNK)
  k = np.arange(WCHUNK)
  blk = (nn[:, None] // 32) == k[None, :]
  w = 2.0 ** (nn % 16)
  slo = (blk * np.where(nn % 32 < 16, w, 0.0)[:, None]).astype(np.float32)
  shi = (blk * np.where(nn % 32 >= 16, w, 0.0)[:, None]).astype(np.float32)
  grp = ((np.arange(WCHUNK)[:, None] // 16) ==
         np.arange(4)[None, :]).astype(np.float32)
  return (jnp.asarray(slo, jnp.bfloat16), jnp.asarray(shi, jnp.bfloat16),
          jnp.asarray(grp, jnp.bfloat16))


def _maskpack_body(q_ref, st_ref, slo_ref, shi_ref, grp_ref, l1_ref, l2_ref,
                   *, n):
  q = q_ref[0]
  qx, qy, qz = q[:, 0:1], q[:, 1:2], q[:, 2:3]
  q2 = (qx * qx + qy * qy) + qz * qz
  qb = q.astype(jnp.bfloat16)
  dn = (((1,), (0,)), ((), ()))
  l2acc = None
  w4 = (jnp.int32(1) << lax.broadcasted_iota(jnp.int32, (1, 4), 1)
        ).astype(jnp.float32)
  for c in range(n // NCHUNK):
    sl = pl.ds(c * NCHUNK, NCHUNK)
    sx = st_ref[0, 0:1, sl]
    sy = st_ref[0, 1:2, sl]
    sz = st_ref[0, 2:3, sl]
    x2 = (sx * sx + sy * sy) + sz * sz
    sb = st_ref[0, :, sl].astype(jnp.bfloat16)
    dot = lax.dot_general(qb, sb, dn, preferred_element_type=jnp.float32)
    d2 = (q2 + x2) - 2.0 * dot
    m = (d2 <= RADIUS2).astype(jnp.bfloat16)
    plo = lax.dot_general(m, slo_ref[...], dn,
                          preferred_element_type=jnp.float32)
    phi = lax.dot_general(m, shi_ref[...], dn,
                          preferred_element_type=jnp.float32)
    comb = plo.astype(jnp.int32) | (phi.astype(jnp.int32) << 16)
    l1_ref[:, pl.ds(c * WCHUNK, WCHUNK)] = comb
    nzw = (comb != 0).astype(jnp.bfloat16)
    gcnt = lax.dot_general(nzw, grp_ref[...], dn,
                           preferred_element_type=jnp.float32)
    gbit = jnp.where(gcnt > 0, w4, 0.0) * (2.0 ** (4 * c))
    contrib = jnp.sum(gbit, axis=1, keepdims=True)
    l2acc = contrib if l2acc is None else l2acc + contrib
  l2_ref[...] = l2acc.astype(jnp.int32)


def _maskpack(query_xyz, support_t):
  b, q, _ = query_xyz.shape
  n = support_t.shape[2]
  qt = 256
  nw = n // 32
  slo, shi, grp = _make_pack_mats()
  body = functools.partial(_maskpack_body, n=n)
  nrow = q // qt
  return pl.pallas_call(
      body,
      grid=(b, nrow),
      in_specs=[
          pl.BlockSpec((1, qt, 3), lambda i, j: (i, j, 0)),
          pl.BlockSpec((1, 3, n), lambda i, j: (i, 0, 0)),
          pl.BlockSpec((NCHUNK, WCHUNK), lambda i, j: (0, 0)),
          pl.BlockSpec((NCHUNK, WCHUNK), lambda i, j: (0, 0)),
          pl.BlockSpec((WCHUNK, 4), lambda i, j: (0, 0)),
      ],
      out_specs=[
          pl.BlockSpec((qt, nw), lambda i, j: (i * nrow + j, 0)),
          pl.BlockSpec((qt, 1), lambda i, j: (i * nrow + j, 0)),
      ],
      out_shape=[
          jax.ShapeDtypeStruct((b * q, nw), jnp.int32),
          jax.ShapeDtypeStruct((b * q, 1), jnp.int32),
      ],
  )(query_xyz, support_t, slo, shi, grp)


def _sc_params():
  cp = pltpu.CompilerParams()
  if "needs_layout_passes" in pltpu.CompilerParams.__dataclass_fields__:
    cp = dataclasses.replace(cp, needs_layout_passes=False)
  return cp


def _iota16():
  return lax.broadcasted_iota(jnp.int32, (L,), 0)


def _splat(x):
  return jnp.broadcast_to(x, (L,))


IDXW = NSAMPLE + 1


def _ballfinish(l1, l2):
  bq, nwords = l1.shape
  per_w = bq // NW
  grp = 64
  mesh = plsc.VectorSubcoreMesh(core_axis_name="c", subcore_axis_name="s")

  @functools.partial(
      pl.kernel,
      out_type=jax.ShapeDtypeStruct((bq * IDXW,), jnp.int32),
      mesh=mesh,
      scratch_types=[
          pltpu.VMEM((grp, nwords), jnp.int32),
          pltpu.VMEM((grp, 1), jnp.int32),
          pltpu.VMEM((L,), jnp.int32),
          pltpu.VMEM((nwords + L,), jnp.int32),
          pltpu.VMEM((64,), jnp.int32),
          pltpu.VMEM((grp * IDXW,), jnp.int32),
      ],
      compiler_params=_sc_params(),
  )
  def k(l1_hbm, l2_hbm, idx_hbm, l1_vm, l2_vm, nzg_vm, nz_vm, hits_vm,
        out_vm):
    wid = lax.axis_index("s") * NC + lax.axis_index("c")
    qbase = wid * per_w
    iota = _iota16()
    zero16 = jnp.zeros((L,), jnp.int32)

    @pl.loop(0, per_w // grp)
    def _group(g):
      qstart = qbase + g * grp
      pltpu.sync_copy(l1_hbm.at[pl.ds(qstart, grp)], l1_vm)
      pltpu.sync_copy(l2_hbm.at[pl.ds(qstart, grp)], l2_vm)

      @pl.loop(0, grp)
      def _query(qi):
        hits_vm[pl.ds(0, L)] = zero16
        l2v = plsc.load_gather(l2_vm, [_splat(qi), zero16])
        gb = (lax.shift_right_logical(l2v, iota) & 1) == 1
        gpfx = plsc.cumsum(gb.astype(jnp.int32))
        gslots = jnp.where(gb, gpfx - 1, 0)
        plsc.store_scatter(nzg_vm, [gslots], iota, mask=gb)
        ngrp = jnp.max(plsc.all_reduce_population_count(gb))

        def grp_body(i, nnz_v):
          gid = plsc.load_gather(nzg_vm, [_splat(i)])
          wvec = plsc.load_gather(l1_vm, [_splat(qi), gid * 16 + iota])
          m = wvec != 0
          pfx = plsc.cumsum(m.astype(jnp.int32))
          slots = jnp.where(m, nnz_v + pfx - 1, 0)
          plsc.store_scatter(nz_vm, [slots], gid * 16 + iota, mask=m)
          return nnz_v + plsc.all_reduce_population_count(m)

        nnz = jnp.max(lax.fori_loop(0, ngrp, grp_body, zero16))

        def word_body(i, hcnt_v):
          kword = plsc.load_gather(nz_vm, [_splat(i)])
          wv = plsc.load_gather(l1_vm, [_splat(qi), kword])
          out = hcnt_v
          for half in range(2):
            bits = lax.shift_right_logical(wv, iota + half * L) & 1
            bm = bits == 1
            pfx = plsc.cumsum(bits)
            slots = out + pfx - 1
            wm = bm & (slots < 48)
            slots = jnp.where(wm, slots, 0)
            ids = kword * 32 + half * L + iota
            plsc.store_scatter(hits_vm, [slots], ids, mask=wm)
            out = out + plsc.all_reduce_population_count(bm)
          return out

        hcnt_v = lax.fori_loop(0, nnz, word_body, zero16)
        m_tot = jnp.minimum(jnp.max(hcnt_v), NSAMPLE)
        sel0 = jnp.where(iota < m_tot, iota, 0)
        sel1 = jnp.where(iota + L < m_tot, iota + L, 0)
        out_vm[pl.ds(qi * IDXW, L)] = plsc.load_gather(hits_vm, [sel0])
        out_vm[pl.ds(qi * IDXW + L, L)] = plsc.load_gather(hits_vm, [sel1])

      pltpu.sync_copy(out_vm, idx_hbm.at[pl.ds(qstart * IDXW, grp * IDXW)])

  return k(l1, l2)


def _gather(features, support_t, query_t, idx):
  b, c, n = features.shape
  q = query_t.shape[2]
  cg = 8
  qch = 128
  sh = NSAMPLE // 2
  mesh = plsc.VectorSubcoreMesh(core_axis_name="c", subcore_axis_name="s")

  @functools.partial(
      pl.kernel,
      out_type=(jax.ShapeDtypeStruct((b, c, NSAMPLE, q), jnp.float32),
                jax.ShapeDtypeStruct((b, 3, NSAMPLE, q), jnp.float32)),
      mesh=mesh,
      scratch_types=[
          pltpu.VMEM((cg, n), jnp.float32),
          pltpu.VMEM((qch * IDXW,), jnp.int32),
          pltpu.VMEM((qch * IDXW,), jnp.int32),
          pltpu.VMEM((1, cg, sh, qch), jnp.float32),
          pltpu.VMEM((1, cg, sh, qch), jnp.float32),
          pltpu.VMEM((4, q), jnp.float32),
          pltpu.SemaphoreType.DMA,
          pltpu.SemaphoreType.DMA,
          pltpu.SemaphoreType.DMA,
          pltpu.SemaphoreType.DMA,
      ],
      compiler_params=_sc_params(),
  )
  def k(f_hbm, st_hbm, qt_hbm, idx_hbm, of_hbm, ox_hbm,
        tab_vm, idx_a, idx_b, out_a, out_b, q_vm, si0, si1, so0, so1):
    wid = lax.axis_index("s") * NC + lax.axis_index("c")
    iota = _iota16()
    si = (si0, si1)
    so = (so0, so1)
    outs = (out_a, out_b)
    idxs = (idx_a, idx_b)

    def idx_cp(tb, s, bi):
      return pltpu.make_async_copy(
          idx_hbm.at[pl.ds((tb * q + s * qch) * IDXW, qch * IDXW)],
          idxs[bi], si[bi])

    def ring(tb, s_lo, s_hi, compute, out_cp):
      idx_cp(tb, s_lo, 0).start()
      idx_cp(tb, s_lo + 1, 1).start()

      @pl.loop(0, (s_hi - s_lo) // 2)
      def _pair(g):
        for bi in range(2):
          s = s_lo + g * 2 + bi
          idx_cp(tb, s, bi).wait()
          for half in range(2):
            @pl.when(s > s_lo)
            def _():
              out_cp(tb, s - 1, half).wait()

            compute(s, bi, half, outs[half])
            out_cp(tb, s, half).start()

          @pl.when(s + 2 < s_hi)
          def _():
            idx_cp(tb, s + 2, bi).start()

      for half in range(2):
        out_cp(tb, s_hi - 1, half).wait()

    def feature_task(tid):
      tb = tid // (c // cg)
      tc = (tid % (c // cg)) * cg
      pltpu.sync_copy(f_hbm.at[tb, pl.ds(tc, cg)], tab_vm)

      def compute(s, bi, half, ob):
        @pl.loop(0, qch // L)
        def _qb(v):
          qrow = (_splat(v * L) + iota) * IDXW
          for s16 in range(sh):
            ivec = plsc.load_gather(idxs[bi], [qrow + half * sh + s16])
            for cc in range(cg):
              g = plsc.load_gather(tab_vm, [_splat(cc), ivec])
              ob[0, cc, s16, pl.ds(v * L, L)] = g

      def out_cp(tb_, s, half):
        return pltpu.make_async_copy(
            outs[half],
            of_hbm.at[pl.ds(tb_, 1), pl.ds(tc, cg),
                      pl.ds(half * sh, sh), pl.ds(s * qch, qch)],
            so[half])

      ring(tb, 0, q // qch, compute, out_cp)

    def xyz_task(xt):
      tb = xt // 4
      h = xt % 4
      nck = q // qch // 4
      pltpu.sync_copy(st_hbm.at[tb], tab_vm.at[pl.ds(0, 3)])
      pltpu.sync_copy(qt_hbm.at[tb], q_vm.at[pl.ds(0, 3)])

      def compute(s, bi, half, ob):
        @pl.loop(0, qch // L)
        def _qb(v):
          qrow = (_splat(v * L) + iota) * IDXW
          qv = [q_vm[d, pl.ds(s * qch + v * L, L)] for d in range(3)]
          for s16 in range(sh):
            ivec = plsc.load_gather(idxs[bi], [qrow + half * sh + s16])
            for d in range(3):
              g = plsc.load_gather(tab_vm, [_splat(d), ivec])
              ob[0, d, s16, pl.ds(v * L, L)] = g - qv[d]

      def out_cp(tb_, s, half):
        return pltpu.make_async_copy(
            outs[half].at[:, pl.ds(0, 3)],
            ox_hbm.at[pl.ds(tb_, 1), :,
                      pl.ds(half * sh, sh), pl.ds(s * qch, qch)], so[half])

      ring(tb, h * nck, (h + 1) * nck, compute, out_cp)

    for r in range(b * (c // cg) // NW):
      feature_task(r * NW + wid)

    xyz_task(wid)

  return k(features, support_t, query_t, idx)


def kernel(query_xyz, support_xyz, features):
  b, q, _ = query_xyz.shape
  n = support_xyz.shape[1]
  c = features.shape[1]
  support_t = jnp.transpose(support_xyz, (0, 2, 1))
  query_t = jnp.transpose(query_xyz, (0, 2, 1))
  h = b // 2
  idx_parts = []
  for lo in (0, h):
    l1, l2 = _maskpack(query_xyz[lo:lo + h], support_t[lo:lo + h])
    idx_parts.append(_ballfinish(l1, l2))
  idx = jnp.concatenate(idx_parts)
  feat_g, xyz_g = _gather(features, support_t, query_t, idx)
  grouped_xyz = jnp.transpose(xyz_g, (0, 1, 3, 2))
  grouped_features = jnp.transpose(feat_g, (0, 1, 3, 2))
  return (grouped_xyz, grouped_features)


# speedup vs baseline: 65.9053x; 1.0654x over previous
import dataclasses
import functools

import jax
import jax.numpy as jnp
import numpy as np
from jax import lax
from jax.experimental import pallas as pl
from jax.experimental.pallas import tpu as pltpu
from jax.experimental.pallas import tpu_sc as plsc

RADIUS2 = 0.25 * 0.25
NSAMPLE = 32

NC = 2
NS = 16
NW = NC * NS
L = 16

NCHUNK = 2048
WCHUNK = NCHUNK // 32


def _make_pack_mats():
  nn = np.arange(NCHUNK)
  k = np.arange(WCHUNK)
  blk = (nn[:, None] // 32) == k[None, :]
  w = 2.0 ** (nn % 16)
  slo = (blk * np.where(nn % 32 < 16, w, 0.0)[:, None]).astype(np.float32)
  shi = (blk * np.where(nn % 32 >= 16, w, 0.0)[:, None]).astype(np.float32)
  grp = ((np.arange(WCHUNK)[:, None] // 16) ==
         np.arange(4)[None, :]).astype(np.float32)
  return (jnp.asarray(slo, jnp.bfloat16), jnp.asarray(shi, jnp.bfloat16),
          jnp.asarray(grp, jnp.bfloat16))


def _maskpack_body(q_ref, st_ref, slo_ref, shi_ref, grp_ref, l1_ref, l2_ref,
                   *, n):
  q = q_ref[0]
  qx, qy, qz = q[:, 0:1], q[:, 1:2], q[:, 2:3]
  q2 = (qx * qx + qy * qy) + qz * qz
  qb = q.astype(jnp.bfloat16)
  dn = (((1,), (0,)), ((), ()))
  l2acc = None
  w4 = (jnp.int32(1) << lax.broadcasted_iota(jnp.int32, (1, 4), 1)
        ).astype(jnp.float32)
  for c in range(n // NCHUNK):
    sl = pl.ds(c * NCHUNK, NCHUNK)
    sx = st_ref[0, 0:1, sl]
    sy = st_ref[0, 1:2, sl]
    sz = st_ref[0, 2:3, sl]
    x2 = (sx * sx + sy * sy) + sz * sz
    sb = st_ref[0, :, sl].astype(jnp.bfloat16)
    dot = lax.dot_general(qb, sb, dn, preferred_element_type=jnp.float32)
    d2 = (q2 + x2) - 2.0 * dot
    m = (d2 <= RADIUS2).astype(jnp.bfloat16)
    plo = lax.dot_general(m, slo_ref[...], dn,
                          preferred_element_type=jnp.float32)
    phi = lax.dot_general(m, shi_ref[...], dn,
                          preferred_element_type=jnp.float32)
    comb = plo.astype(jnp.int32) | (phi.astype(jnp.int32) << 16)
    l1_ref[:, pl.ds(c * WCHUNK, WCHUNK)] = comb
    nzw = (comb != 0).astype(jnp.bfloat16)
    gcnt = lax.dot_general(nzw, grp_ref[...], dn,
                           preferred_element_type=jnp.float32)
    gbit = jnp.where(gcnt > 0, w4, 0.0) * (2.0 ** (4 * c))
    contrib = jnp.sum(gbit, axis=1, keepdims=True)
    l2acc = contrib if l2acc is None else l2acc + contrib
  l2_ref[...] = l2acc.astype(jnp.int32)


def _maskpack(query_xyz, support_t):
  b, q, _ = query_xyz.shape
  n = support_t.shape[2]
  qt = 256
  nw = n // 32
  slo, shi, grp = _make_pack_mats()
  body = functools.partial(_maskpack_body, n=n)
  nrow = q // qt
  return pl.pallas_call(
      body,
      grid=(b, nrow),
      in_specs=[
          pl.BlockSpec((1, qt, 3), lambda i, j: (i, j, 0)),
          pl.BlockSpec((1, 3, n), lambda i, j: (i, 0, 0)),
          pl.BlockSpec((NCHUNK, WCHUNK), lambda i, j: (0, 0)),
          pl.BlockSpec((NCHUNK, WCHUNK), lambda i, j: (0, 0)),
          pl.BlockSpec((WCHUNK, 4), lambda i, j: (0, 0)),
      ],
      out_specs=[
          pl.BlockSpec((qt, nw), lambda i, j: (i * nrow + j, 0)),
          pl.BlockSpec((qt, 1), lambda i, j: (i * nrow + j, 0)),
      ],
      out_shape=[
          jax.ShapeDtypeStruct((b * q, nw), jnp.int32),
          jax.ShapeDtypeStruct((b * q, 1), jnp.int32),
      ],
  )(query_xyz, support_t, slo, shi, grp)


def _sc_params():
  cp = pltpu.CompilerParams()
  if "needs_layout_passes" in pltpu.CompilerParams.__dataclass_fields__:
    cp = dataclasses.replace(cp, needs_layout_passes=False)
  return cp


def _iota16():
  return lax.broadcasted_iota(jnp.int32, (L,), 0)


def _splat(x):
  return jnp.broadcast_to(x, (L,))


IDXW = NSAMPLE + 1


def _ballfinish(l1, l2):
  bq, nwords = l1.shape
  per_w = bq // NW
  grp = 64
  mesh = plsc.VectorSubcoreMesh(core_axis_name="c", subcore_axis_name="s")

  @functools.partial(
      pl.kernel,
      out_type=jax.ShapeDtypeStruct((bq * IDXW,), jnp.int32),
      mesh=mesh,
      scratch_types=[
          pltpu.VMEM((grp, nwords), jnp.int32),
          pltpu.VMEM((grp, 1), jnp.int32),
          pltpu.VMEM((L,), jnp.int32),
          pltpu.VMEM((nwords + L,), jnp.int32),
          pltpu.VMEM((64,), jnp.int32),
          pltpu.VMEM((grp * IDXW,), jnp.int32),
      ],
      compiler_params=_sc_params(),
  )
  def k(l1_hbm, l2_hbm, idx_hbm, l1_vm, l2_vm, nzg_vm, nz_vm, hits_vm,
        out_vm):
    wid = lax.axis_index("s") * NC + lax.axis_index("c")
    qbase = wid * per_w
    iota = _iota16()
    zero16 = jnp.zeros((L,), jnp.int32)

    @pl.loop(0, per_w // grp)
    def _group(g):
      qstart = qbase + g * grp
      pltpu.sync_copy(l1_hbm.at[pl.ds(qstart, grp)], l1_vm)
      pltpu.sync_copy(l2_hbm.at[pl.ds(qstart, grp)], l2_vm)

      @pl.loop(0, grp)
      def _query(qi):
        hits_vm[pl.ds(0, L)] = zero16
        l2v = plsc.load_gather(l2_vm, [_splat(qi), zero16])
        gb = (lax.shift_right_logical(l2v, iota) & 1) == 1
        gpfx = plsc.cumsum(gb.astype(jnp.int32))
        gslots = jnp.where(gb, gpfx - 1, 0)
        plsc.store_scatter(nzg_vm, [gslots], iota, mask=gb)
        ngrp = jnp.max(plsc.all_reduce_population_count(gb))

        def grp_body(i, nnz_v):
          gid = plsc.load_gather(nzg_vm, [_splat(i)])
          wvec = plsc.load_gather(l1_vm, [_splat(qi), gid * 16 + iota])
          m = wvec != 0
          pfx = plsc.cumsum(m.astype(jnp.int32))
          slots = jnp.where(m, nnz_v + pfx - 1, 0)
          plsc.store_scatter(nz_vm, [slots], gid * 16 + iota, mask=m)
          return nnz_v + plsc.all_reduce_population_count(m)

        nnz = jnp.max(lax.fori_loop(0, ngrp, grp_body, zero16))

        def word_body(i, hcnt_v):
          kword = plsc.load_gather(nz_vm, [_splat(i)])
          wv = plsc.load_gather(l1_vm, [_splat(qi), kword])
          out = hcnt_v
          for half in range(2):
            bits = lax.shift_right_logical(wv, iota + half * L) & 1
            bm = bits == 1
            pfx = plsc.cumsum(bits)
            slots = out + pfx - 1
            wm = bm & (slots < 48)
            slots = jnp.where(wm, slots, 0)
            ids = kword * 32 + half * L + iota
            plsc.store_scatter(hits_vm, [slots], ids, mask=wm)
            out = out + plsc.all_reduce_population_count(bm)
          return out

        hcnt_v = lax.fori_loop(0, nnz, word_body, zero16)
        m_tot = jnp.minimum(jnp.max(hcnt_v), NSAMPLE)
        sel0 = jnp.where(iota < m_tot, iota, 0)
        sel1 = jnp.where(iota + L < m_tot, iota + L, 0)
        out_vm[pl.ds(qi * IDXW, L)] = plsc.load_gather(hits_vm, [sel0])
        out_vm[pl.ds(qi * IDXW + L, L)] = plsc.load_gather(hits_vm, [sel1])

      pltpu.sync_copy(out_vm, idx_hbm.at[pl.ds(qstart * IDXW, grp * IDXW)])

  return k(l1, l2)


def _gather(features, support_t, query_t, idx):
  b, c, n = features.shape
  q = query_t.shape[2]
  cg = 8
  qch = 128
  sh = NSAMPLE // 2
  mesh = plsc.VectorSubcoreMesh(core_axis_name="c", subcore_axis_name="s")

  @functools.partial(
      pl.kernel,
      out_type=(jax.ShapeDtypeStruct((b, c, NSAMPLE, q), jnp.float32),
                jax.ShapeDtypeStruct((b, 3, NSAMPLE, q), jnp.float32)),
      mesh=mesh,
      scratch_types=[
          pltpu.VMEM((cg, n), jnp.float32),
          pltpu.VMEM((qch * IDXW,), jnp.int32),
          pltpu.VMEM((qch * IDXW,), jnp.int32),
          pltpu.VMEM((1, cg, sh, qch), jnp.float32),
          pltpu.VMEM((1, cg, sh, qch), jnp.float32),
          pltpu.VMEM((4, q), jnp.float32),
          pltpu.SemaphoreType.DMA,
          pltpu.SemaphoreType.DMA,
          pltpu.SemaphoreType.DMA,
          pltpu.SemaphoreType.DMA,
      ],
      compiler_params=_sc_params(),
  )
  def k(f_hbm, st_hbm, qt_hbm, idx_hbm, of_hbm, ox_hbm,
        tab_vm, idx_a, idx_b, out_a, out_b, q_vm, si0, si1, so0, so1):
    wid = lax.axis_index("s") * NC + lax.axis_index("c")
    iota = _iota16()
    si = (si0, si1)
    so = (so0, so1)
    outs = (out_a, out_b)
    idxs = (idx_a, idx_b)

    def idx_cp(tb, s, bi):
      return pltpu.make_async_copy(
          idx_hbm.at[pl.ds((tb * q + s * qch) * IDXW, qch * IDXW)],
          idxs[bi], si[bi])

    def ring(tb, s_lo, s_hi, compute, out_cp):
      idx_cp(tb, s_lo, 0).start()
      idx_cp(tb, s_lo + 1, 1).start()

      @pl.loop(0, (s_hi - s_lo) // 2)
      def _pair(g):
        for bi in range(2):
          s = s_lo + g * 2 + bi
          idx_cp(tb, s, bi).wait()
          for half in range(2):
            @pl.when(s > s_lo)
            def _():
              out_cp(tb, s - 1, half).wait()

            compute(s, bi, half, outs[half])
            out_cp(tb, s, half).start()

          @pl.when(s + 2 < s_hi)
          def _():
            idx_cp(tb, s + 2, bi).start()

      for half in range(2):
        out_cp(tb, s_hi - 1, half).wait()

    def feature_task(tid):
      tb = tid // (c // cg)
      tc = (tid % (c // cg)) * cg
      pltpu.sync_copy(f_hbm.at[tb, pl.ds(tc, cg)], tab_vm)

      def compute(s, bi, half, ob):
        @pl.loop(0, qch // L)
        def _qb(v):
          qrow = (_splat(v * L) + iota) * IDXW
          for s16 in range(sh):
            ivec = plsc.load_gather(idxs[bi], [qrow + half * sh + s16])
            for cc in range(cg):
              g = plsc.load_gather(tab_vm, [_splat(cc), ivec])
              ob[0, cc, s16, pl.ds(v * L, L)] = g

      def out_cp(tb_, s, half):
        return pltpu.make_async_copy(
            outs[half],
            of_hbm.at[pl.ds(tb_, 1), pl.ds(tc, cg),
                      pl.ds(half * sh, sh), pl.ds(s * qch, qch)],
            so[half])

      ring(tb, 0, q // qch, compute, out_cp)

    def xyz_task(xt):
      tb = xt // 4
      h = xt % 4
      nck = q // qch // 4
      pltpu.sync_copy(st_hbm.at[tb], tab_vm.at[pl.ds(0, 3)])
      pltpu.sync_copy(qt_hbm.at[tb], q_vm.at[pl.ds(0, 3)])

      def compute(s, bi, half, ob):
        @pl.loop(0, qch // L)
        def _qb(v):
          qrow = (_splat(v * L) + iota) * IDXW
          qv = [q_vm[d, pl.ds(s * qch + v * L, L)] for d in range(3)]
          for s16 in range(sh):
            ivec = plsc.load_gather(idxs[bi], [qrow + half * sh + s16])
            for d in range(3):
              g = plsc.load_gather(tab_vm, [_splat(d), ivec])
              ob[0, d, s16, pl.ds(v * L, L)] = g - qv[d]

      def out_cp(tb_, s, half):
        return pltpu.make_async_copy(
            outs[half].at[:, pl.ds(0, 3)],
            ox_hbm.at[pl.ds(tb_, 1), :,
                      pl.ds(half * sh, sh), pl.ds(s * qch, qch)], so[half])

      ring(tb, h * nck, (h + 1) * nck, compute, out_cp)

    for r in range(b * (c // cg) // NW):
      feature_task(r * NW + wid)

    xyz_task(wid)

  return k(features, support_t, query_t, idx)


def kernel(query_xyz, support_xyz, features):
  b, q, _ = query_xyz.shape
  n = support_xyz.shape[1]
  c = features.shape[1]
  support_t = jnp.transpose(support_xyz, (0, 2, 1))
  query_t = jnp.transpose(query_xyz, (0, 2, 1))
  h = b // 4
  idx_parts = []
  for lo in range(0, b, h):
    l1, l2 = _maskpack(query_xyz[lo:lo + h], support_t[lo:lo + h])
    idx_parts.append(_ballfinish(l1, l2))
  idx = jnp.concatenate(idx_parts)
  feat_g, xyz_g = _gather(features, support_t, query_t, idx)
  grouped_xyz = jnp.transpose(xyz_g, (0, 1, 3, 2))
  grouped_features = jnp.transpose(feat_g, (0, 1, 3, 2))
  return (grouped_xyz, grouped_features)


# speedup vs baseline: 107.7186x; 1.6344x over previous
import dataclasses
import functools

import jax
import jax.numpy as jnp
import numpy as np
from jax import lax
from jax.experimental import pallas as pl
from jax.experimental.pallas import tpu as pltpu
from jax.experimental.pallas import tpu_sc as plsc

RADIUS2 = 0.25 * 0.25
NSAMPLE = 32

NC = 2
NS = 16
NW = NC * NS
L = 16

NCHUNK = 2048
WCHUNK = NCHUNK // 32


def _make_pack_mats():
  nn = np.arange(NCHUNK)
  k = np.arange(WCHUNK)
  blk = (nn[:, None] // 32) == k[None, :]
  w = 2.0 ** (nn % 16)
  slo = (blk * np.where(nn % 32 < 16, w, 0.0)[:, None]).astype(np.float32)
  shi = (blk * np.where(nn % 32 >= 16, w, 0.0)[:, None]).astype(np.float32)
  grp = ((np.arange(WCHUNK)[:, None] // 16) ==
         np.arange(4)[None, :]).astype(np.float32)
  return (jnp.asarray(slo, jnp.bfloat16), jnp.asarray(shi, jnp.bfloat16),
          jnp.asarray(grp, jnp.bfloat16))


def _maskpack_body(q_ref, st_ref, slo_ref, shi_ref, grp_ref, l1_ref, l2_ref,
                   *, n):
  q = q_ref[0]
  qx, qy, qz = q[:, 0:1], q[:, 1:2], q[:, 2:3]
  q2 = (qx * qx + qy * qy) + qz * qz
  qb = q.astype(jnp.bfloat16)
  dn = (((1,), (0,)), ((), ()))
  l2acc = None
  w4 = (jnp.int32(1) << lax.broadcasted_iota(jnp.int32, (1, 4), 1)
        ).astype(jnp.float32)
  for c in range(n // NCHUNK):
    sl = pl.ds(c * NCHUNK, NCHUNK)
    sx = st_ref[0, 0:1, sl]
    sy = st_ref[0, 1:2, sl]
    sz = st_ref[0, 2:3, sl]
    x2 = (sx * sx + sy * sy) + sz * sz
    sb = st_ref[0, :, sl].astype(jnp.bfloat16)
    dot = lax.dot_general(qb, sb, dn, preferred_element_type=jnp.float32)
    d2 = (q2 + x2) - 2.0 * dot
    m = (d2 <= RADIUS2).astype(jnp.bfloat16)
    plo = lax.dot_general(m, slo_ref[...], dn,
                          preferred_element_type=jnp.float32)
    phi = lax.dot_general(m, shi_ref[...], dn,
                          preferred_element_type=jnp.float32)
    comb = plo.astype(jnp.int32) | (phi.astype(jnp.int32) << 16)
    l1_ref[:, pl.ds(c * WCHUNK, WCHUNK)] = comb
    nzw = (comb != 0).astype(jnp.bfloat16)
    gcnt = lax.dot_general(nzw, grp_ref[...], dn,
                           preferred_element_type=jnp.float32)
    gbit = jnp.where(gcnt > 0, w4, 0.0) * (2.0 ** (4 * c))
    contrib = jnp.sum(gbit, axis=1, keepdims=True)
    l2acc = contrib if l2acc is None else l2acc + contrib
  l2_ref[...] = l2acc.astype(jnp.int32)


def _maskpack(query_xyz, support_t):
  b, q, _ = query_xyz.shape
  n = support_t.shape[2]
  qt = 256
  nw = n // 32
  slo, shi, grp = _make_pack_mats()
  body = functools.partial(_maskpack_body, n=n)
  nrow = q // qt
  return pl.pallas_call(
      body,
      grid=(b, nrow),
      in_specs=[
          pl.BlockSpec((1, qt, 3), lambda i, j: (i, j, 0)),
          pl.BlockSpec((1, 3, n), lambda i, j: (i, 0, 0)),
          pl.BlockSpec((NCHUNK, WCHUNK), lambda i, j: (0, 0)),
          pl.BlockSpec((NCHUNK, WCHUNK), lambda i, j: (0, 0)),
          pl.BlockSpec((WCHUNK, 4), lambda i, j: (0, 0)),
      ],
      out_specs=[
          pl.BlockSpec((qt, nw), lambda i, j: (i * nrow + j, 0)),
          pl.BlockSpec((qt, 1), lambda i, j: (i * nrow + j, 0)),
      ],
      out_shape=[
          jax.ShapeDtypeStruct((b * q, nw), jnp.int32),
          jax.ShapeDtypeStruct((b * q, 1), jnp.int32),
      ],
  )(query_xyz, support_t, slo, shi, grp)


def _sc_params():
  cp = pltpu.CompilerParams()
  if "needs_layout_passes" in pltpu.CompilerParams.__dataclass_fields__:
    cp = dataclasses.replace(cp, needs_layout_passes=False)
  return cp


def _iota16():
  return lax.broadcasted_iota(jnp.int32, (L,), 0)


def _splat(x):
  return jnp.broadcast_to(x, (L,))


IDXW = NSAMPLE + 1


def _ballfinish(l1, l2):
  bq, nwords = l1.shape
  per_w = bq // NW
  grp = 64
  mesh = plsc.VectorSubcoreMesh(core_axis_name="c", subcore_axis_name="s")

  @functools.partial(
      pl.kernel,
      out_type=jax.ShapeDtypeStruct((bq * IDXW,), jnp.int32),
      mesh=mesh,
      scratch_types=[
          pltpu.VMEM((grp, nwords), jnp.int32),
          pltpu.VMEM((grp, 1), jnp.int32),
          pltpu.VMEM((L,), jnp.int32),
          pltpu.VMEM((nwords + L,), jnp.int32),
          pltpu.VMEM((64,), jnp.int32),
          pltpu.VMEM((grp * IDXW,), jnp.int32),
      ],
      compiler_params=_sc_params(),
  )
  def k(l1_hbm, l2_hbm, idx_hbm, l1_vm, l2_vm, nzg_vm, nz_vm, hits_vm,
        out_vm):
    wid = lax.axis_index("s") * NC + lax.axis_index("c")
    qbase = wid * per_w
    iota = _iota16()
    zero16 = jnp.zeros((L,), jnp.int32)

    @pl.loop(0, per_w // grp)
    def _group(g):
      qstart = qbase + g * grp
      pltpu.sync_copy(l1_hbm.at[pl.ds(qstart, grp)], l1_vm)
      pltpu.sync_copy(l2_hbm.at[pl.ds(qstart, grp)], l2_vm)

      @pl.loop(0, grp)
      def _query(qi):
        hits_vm[pl.ds(0, L)] = zero16
        l2v = plsc.load_gather(l2_vm, [_splat(qi), zero16])
        gb = (lax.shift_right_logical(l2v, iota) & 1) == 1
        gpfx = plsc.cumsum(gb.astype(jnp.int32))
        gslots = jnp.where(gb, gpfx - 1, 0)
        plsc.store_scatter(nzg_vm, [gslots], iota, mask=gb)
        ngrp = jnp.max(plsc.all_reduce_population_count(gb))

        def grp_body(i, nnz_v):
          gid = plsc.load_gather(nzg_vm, [_splat(i)])
          wvec = plsc.load_gather(l1_vm, [_splat(qi), gid * 16 + iota])
          m = wvec != 0
          pfx = plsc.cumsum(m.astype(jnp.int32))
          slots = jnp.where(m, nnz_v + pfx - 1, 0)
          plsc.store_scatter(nz_vm, [slots], gid * 16 + iota, mask=m)
          return nnz_v + plsc.all_reduce_population_count(m)

        nnz = jnp.max(lax.fori_loop(0, ngrp, grp_body, zero16))

        def word_body(i, hcnt_v):
          kword = plsc.load_gather(nz_vm, [_splat(i)])
          wv = plsc.load_gather(l1_vm, [_splat(qi), kword])
          out = hcnt_v
          for half in range(2):
            bits = lax.shift_right_logical(wv, iota + half * L) & 1
            bm = bits == 1
            pfx = plsc.cumsum(bits)
            slots = out + pfx - 1
            wm = bm & (slots < 48)
            slots = jnp.where(wm, slots, 0)
            ids = kword * 32 + half * L + iota
            plsc.store_scatter(hits_vm, [slots], ids, mask=wm)
            out = out + plsc.all_reduce_population_count(bm)
          return out

        hcnt_v = lax.fori_loop(0, nnz, word_body, zero16)
        m_tot = jnp.minimum(jnp.max(hcnt_v), NSAMPLE)
        sel0 = jnp.where(iota < m_tot, iota, 0)
        sel1 = jnp.where(iota + L < m_tot, iota + L, 0)
        out_vm[pl.ds(qi * IDXW, L)] = plsc.load_gather(hits_vm, [sel0])
        out_vm[pl.ds(qi * IDXW + L, L)] = plsc.load_gather(hits_vm, [sel1])

      pltpu.sync_copy(out_vm, idx_hbm.at[pl.ds(qstart * IDXW, grp * IDXW)])

  return k(l1, l2)


def _gather(features, support_t, query_t, idx):
  b, c, n = features.shape
  q = query_t.shape[2]
  cg = 8
  qch = 128
  sh = NSAMPLE // 2
  mesh = plsc.VectorSubcoreMesh(core_axis_name="c", subcore_axis_name="s")

  @functools.partial(
      pl.kernel,
      out_type=(jax.ShapeDtypeStruct((b, c, NSAMPLE, q), jnp.float32),
                jax.ShapeDtypeStruct((b, 3, NSAMPLE, q), jnp.float32)),
      mesh=mesh,
      scratch_types=[
          pltpu.VMEM((cg, n), jnp.float32),
          pltpu.VMEM((qch * IDXW,), jnp.int32),
          pltpu.VMEM((qch * IDXW,), jnp.int32),
          pltpu.VMEM((1, cg, sh, qch), jnp.float32),
          pltpu.VMEM((1, cg, sh, qch), jnp.float32),
          pltpu.VMEM((4, q), jnp.float32),
          pltpu.SemaphoreType.DMA,
          pltpu.SemaphoreType.DMA,
          pltpu.SemaphoreType.DMA,
          pltpu.SemaphoreType.DMA,
      ],
      compiler_params=_sc_params(),
  )
  def k(f_hbm, st_hbm, qt_hbm, idx_hbm, of_hbm, ox_hbm,
        tab_vm, idx_a, idx_b, out_a, out_b, q_vm, si0, si1, so0, so1):
    wid = lax.axis_index("s") * NC + lax.axis_index("c")
    iota = _iota16()
    si = (si0, si1)
    so = (so0, so1)
    outs = (out_a, out_b)
    idxs = (idx_a, idx_b)

    def idx_cp(tb, s, bi):
      return pltpu.make_async_copy(
          idx_hbm.at[pl.ds((tb * q + s * qch) * IDXW, qch * IDXW)],
          idxs[bi], si[bi])

    def ring(tb, s_lo, s_hi, compute, out_cp):
      idx_cp(tb, s_lo, 0).start()
      idx_cp(tb, s_lo + 1, 1).start()

      @pl.loop(0, (s_hi - s_lo) // 2)
      def _pair(g):
        for bi in range(2):
          s = s_lo + g * 2 + bi
          idx_cp(tb, s, bi).wait()
          for half in range(2):
            @pl.when(s > s_lo)
            def _():
              out_cp(tb, s - 1, half).wait()

            compute(s, bi, half, outs[half])
            out_cp(tb, s, half).start()

          @pl.when(s + 2 < s_hi)
          def _():
            idx_cp(tb, s + 2, bi).start()

      for half in range(2):
        out_cp(tb, s_hi - 1, half).wait()

    def feature_task(tid):
      tb = tid // (c // cg)
      tc = (tid % (c // cg)) * cg
      pltpu.sync_copy(f_hbm.at[tb, pl.ds(tc, cg)], tab_vm)

      def compute(s, bi, half, ob):
        @functools.partial(plsc.parallel_loop, 0, qch // L, unroll=2)
        def _qb(v):
          qrow = (_splat(v * L) + iota) * IDXW
          for s16 in range(sh):
            ivec = plsc.load_gather(idxs[bi], [qrow + half * sh + s16])
            for cc in range(cg):
              g = plsc.load_gather(tab_vm, [_splat(cc), ivec])
              ob[0, cc, s16, pl.ds(v * L, L)] = g

      def out_cp(tb_, s, half):
        return pltpu.make_async_copy(
            outs[half],
            of_hbm.at[pl.ds(tb_, 1), pl.ds(tc, cg),
                      pl.ds(half * sh, sh), pl.ds(s * qch, qch)],
            so[half])

      ring(tb, 0, q // qch, compute, out_cp)

    def xyz_task(xt):
      tb = xt // 4
      h = xt % 4
      nck = q // qch // 4
      pltpu.sync_copy(st_hbm.at[tb], tab_vm.at[pl.ds(0, 3)])
      pltpu.sync_copy(qt_hbm.at[tb], q_vm.at[pl.ds(0, 3)])

      def compute(s, bi, half, ob):
        @functools.partial(plsc.parallel_loop, 0, qch // L, unroll=2)
        def _qb(v):
          qrow = (_splat(v * L) + iota) * IDXW
          qv = [q_vm[d, pl.ds(s * qch + v * L, L)] for d in range(3)]
          for s16 in range(sh):
            ivec = plsc.load_gather(idxs[bi], [qrow + half * sh + s16])
            for d in range(3):
              g = plsc.load_gather(tab_vm, [_splat(d), ivec])
              ob[0, d, s16, pl.ds(v * L, L)] = g - qv[d]

      def out_cp(tb_, s, half):
        return pltpu.make_async_copy(
            outs[half].at[:, pl.ds(0, 3)],
            ox_hbm.at[pl.ds(tb_, 1), :,
                      pl.ds(half * sh, sh), pl.ds(s * qch, qch)], so[half])

      ring(tb, h * nck, (h + 1) * nck, compute, out_cp)

    for r in range(b * (c // cg) // NW):
      feature_task(r * NW + wid)

    xyz_task(wid)

  return k(features, support_t, query_t, idx)


def kernel(query_xyz, support_xyz, features):
  b, q, _ = query_xyz.shape
  n = support_xyz.shape[1]
  c = features.shape[1]
  support_t = jnp.transpose(support_xyz, (0, 2, 1))
  query_t = jnp.transpose(query_xyz, (0, 2, 1))
  h = b // 4
  idx_parts = []
  for lo in range(0, b, h):
    l1, l2 = _maskpack(query_xyz[lo:lo + h], support_t[lo:lo + h])
    idx_parts.append(_ballfinish(l1, l2))
  idx = jnp.concatenate(idx_parts)
  feat_g, xyz_g = _gather(features, support_t, query_t, idx)
  grouped_xyz = jnp.transpose(xyz_g, (0, 1, 3, 2))
  grouped_features = jnp.transpose(feat_g, (0, 1, 3, 2))
  return (grouped_xyz, grouped_features)
